# Initial kernel scaffold; baseline (speedup 1.0000x reference)
import jax
import jax.numpy as jnp
from jax.experimental import pallas as pl


def kernel(x, edge_index, batch, W0, b0, W1, b1, W2, b2, W3, b3, cw1, cb1, cw2, cb2, dw1, db1, dw2, db2):
    raise NotImplementedError("write your pallas kernel here")


# speedup vs baseline: 14.7408x; 14.7408x over previous
import functools

import jax
import jax.numpy as jnp
from jax import lax
from jax.experimental import pallas as pl
from jax.experimental.pallas import tpu as pltpu
from jax.experimental.pallas import tpu_sc as plsc

N = 10000
NP = 10240
E = 320000
F = 32
D = 128
NC, NS, L = 2, 16, 16
NW = NC * NS
CH = 128
CPT = 79
EP = CPT * NW * CH
RPS = NP // NS
KROWS = NP // L
B = 64
K = 30

_mesh = plsc.VectorSubcoreMesh(core_axis_name="c", subcore_axis_name="s",
                               num_cores=NC, num_subcores=NS)
_sc_params = pltpu.CompilerParams(use_tc_tiling_on_sc=False,
                                  needs_layout_passes=False)


def _iota16():
    return lax.broadcasted_iota(jnp.int32, (L,), 0)


def _agg_body(with_deg):
    def body(xl_hbm, src_hbm, dst_hbm, z32_hbm, *rest):
        if with_deg:
            (z16_hbm, p_hbm, pd_hbm, acc, sbuf, dbuf, rows, sem, accd,
             ones) = rest
        else:
            p_hbm, acc, sbuf, dbuf, rows, sem = rest
        c = lax.axis_index("c")
        s = lax.axis_index("s")
        wid = c * NS + s
        r0 = s * RPS
        pltpu.sync_copy(z32_hbm.at[pl.ds(r0, RPS)], acc.at[pl.ds(r0, RPS)])
        if with_deg:
            pltpu.sync_copy(z16_hbm.at[pl.ds(r0, RPS)], accd.at[pl.ds(r0, RPS)])

            def oinit(r, _):
                ones[r, :] = jnp.ones((L,), jnp.float32)
                return 0

            lax.fori_loop(0, CH, oinit, 0)
        plsc.subcore_barrier()

        def chunk(k, _):
            off = (k * NW + wid) * CH
            pltpu.sync_copy(src_hbm.at[pl.ds(off, CH)], sbuf)
            pltpu.sync_copy(dst_hbm.at[pl.ds(off, CH)], dbuf)
            pltpu.async_copy(xl_hbm.at[sbuf], rows, sem).wait()
            pltpu.sync_copy(rows, acc.at[dbuf], add=True)
            if with_deg:
                pltpu.sync_copy(ones, accd.at[sbuf], add=True)
            return 0

        lax.fori_loop(0, CPT, chunk, 0)
        plsc.subcore_barrier()
        pltpu.sync_copy(acc.at[pl.ds(r0, RPS)], p_hbm.at[c, pl.ds(r0, RPS)])
        if with_deg:
            pltpu.sync_copy(accd.at[pl.ds(r0, RPS)],
                            pd_hbm.at[c, pl.ds(r0, RPS)])

    return body


_agg_scratch = [
    pltpu.VMEM_SHARED((NP, F), jnp.float32),
    pltpu.VMEM((CH,), jnp.int32),
    pltpu.VMEM((CH,), jnp.int32),
    pltpu.VMEM((CH, F), jnp.float32),
    pltpu.SemaphoreType.DMA,
]

_agg_deg = pl.kernel(
    _agg_body(True),
    out_type=(jax.ShapeDtypeStruct((NC, NP, F), jnp.float32),
              jax.ShapeDtypeStruct((NC, NP, L), jnp.float32)),
    mesh=_mesh,
    compiler_params=_sc_params,
    scratch_types=_agg_scratch + [pltpu.VMEM_SHARED((NP, L), jnp.float32),
                                  pltpu.VMEM((CH, L), jnp.float32)],
)

_agg = pl.kernel(
    _agg_body(False),
    out_type=jax.ShapeDtypeStruct((NC, NP, F), jnp.float32),
    mesh=_mesh,
    compiler_params=_sc_params,
    scratch_types=_agg_scratch,
)


def _pool_body(keys_hbm, batch_hbm, xcat_hbm, out_hbm, kbuf, bbuf, idxbuf,
               rows, sem):
    c = lax.axis_index("c")
    s = lax.axis_index("s")
    wid = c * NS + s
    pltpu.sync_copy(keys_hbm, kbuf)
    pltpu.sync_copy(batch_hbm, bbuf)
    it = _iota16()
    lane0 = it == 0
    neginf = jnp.full((L,), -jnp.inf, jnp.float32)
    for gi in range(2):
        g = 2 * wid + gi

        def cstep(r, carry):
            bc, bs = carry
            bvec = bbuf[r]
            bc = bc + jnp.where(bvec == g, 1, 0).astype(jnp.int32)
            bs = bs + jnp.where(bvec < g, 1, 0).astype(jnp.int32)
            return bc, bs

        zeros16i = jnp.zeros((L,), jnp.int32)
        bc, bs = lax.fori_loop(0, KROWS, cstep, (zeros16i, zeros16i))
        cnt = jnp.sum(bc)
        start = jnp.sum(bs)
        end = start + cnt
        rlo = start // L
        rhi = (end + L - 1) // L
        dum = N + ((it + 2 * wid) & 15)
        idxbuf[pl.ds(0, L)] = dum
        idxbuf[pl.ds(L, L)] = dum

        def tstep(t, _):
            def rstep(r, vc):
                vb, ib = vc
                v = kbuf[r]
                e = r * L + it
                vm = jnp.where((e >= start) & (e < end), v, -jnp.inf)
                upd = vm > vb
                return jnp.where(upd, vm, vb), jnp.where(upd, e, ib)

            vb, ib = lax.fori_loop(rlo, rhi, rstep,
                                   (neginf, jnp.full((L,), 2**30, jnp.int32)))
            gmax = jnp.max(vb)
            sel = jnp.min(jnp.where(vb == gmax, ib, 2**30))
            valid = t < cnt
            node = jnp.where(valid, sel, N + ((2 * wid + t) & 15))
            plsc.store_scatter(idxbuf, [jnp.broadcast_to(t, (L,))],
                               jnp.broadcast_to(node, (L,)), mask=lane0)
            plsc.store_scatter(
                kbuf,
                [jnp.broadcast_to(sel // L, (L,)),
                 jnp.broadcast_to(sel % L, (L,))],
                neginf, mask=lane0 & jnp.broadcast_to(valid, (L,)))
            return 0

        lax.fori_loop(0, K, tstep, 0)
        pltpu.async_copy(xcat_hbm.at[idxbuf], rows, sem).wait()

        def zstep(t, _):
            for cc in range(8):
                rows[t, pl.ds(cc * L, L)] = jnp.zeros((L,), jnp.float32)
            return 0

        lax.fori_loop(jnp.minimum(cnt, K), K, zstep, 0)
        pltpu.sync_copy(rows.at[pl.ds(0, K)], out_hbm.at[g])


_pool = pl.kernel(
    _pool_body,
    out_type=jax.ShapeDtypeStruct((B, K, D), jnp.float32),
    mesh=_mesh,
    compiler_params=_sc_params,
    scratch_types=[
        pltpu.VMEM((KROWS, L), jnp.float32),
        pltpu.VMEM((KROWS, L), jnp.int32),
        pltpu.VMEM((2 * L,), jnp.int32),
        pltpu.VMEM((2 * L, D), jnp.float32),
        pltpu.SemaphoreType.DMA,
    ],
)


_GRID = 8
_BR = NP // _GRID


def _k0_body(x_ref, w_ref, b_ref, o_ref):
    o_ref[...] = jnp.dot(x_ref[...], w_ref[...],
                         preferred_element_type=jnp.float32) + b_ref[...]


_k0 = pl.pallas_call(
    _k0_body,
    grid=(_GRID,),
    in_specs=[
        pl.BlockSpec((_BR, D), lambda i: (i, 0)),
        pl.BlockSpec((D, F), lambda i: (0, 0)),
        pl.BlockSpec((1, F), lambda i: (0, 0)),
    ],
    out_specs=pl.BlockSpec((_BR, F), lambda i: (i, 0)),
    out_shape=jax.ShapeDtypeStruct((NP, F), jnp.float32),
)


def _klayer_body(p_ref, pd_ref, xl_ref, w_ref, b_ref, h_ref, o_ref):
    pd = pd_ref[0] + pd_ref[1]
    dinv = 1.0 / (pd[:, 0:1] + 1.0)
    h = jnp.tanh((p_ref[0] + p_ref[1] + xl_ref[...]) * dinv)
    h_ref[...] = h
    o_ref[...] = jnp.dot(h, w_ref[...],
                         preferred_element_type=jnp.float32) + b_ref[...]


_klayer = pl.pallas_call(
    _klayer_body,
    grid=(_GRID,),
    in_specs=[
        pl.BlockSpec((NC, _BR, F), lambda i: (0, i, 0)),
        pl.BlockSpec((NC, _BR, L), lambda i: (0, i, 0)),
        pl.BlockSpec((_BR, F), lambda i: (i, 0)),
        pl.BlockSpec((F, F), lambda i: (0, 0)),
        pl.BlockSpec((1, F), lambda i: (0, 0)),
    ],
    out_specs=[
        pl.BlockSpec((_BR, F), lambda i: (i, 0)),
        pl.BlockSpec((_BR, F), lambda i: (i, 0)),
    ],
    out_shape=[jax.ShapeDtypeStruct((NP, F), jnp.float32),
               jax.ShapeDtypeStruct((NP, F), jnp.float32)],
)


def _k4_body(p_ref, pd_ref, xl_ref, h_ref):
    pd = pd_ref[0] + pd_ref[1]
    dinv = 1.0 / (pd[:, 0:1] + 1.0)
    h_ref[...] = jnp.tanh((p_ref[0] + p_ref[1] + xl_ref[...]) * dinv)


_k4 = pl.pallas_call(
    _k4_body,
    grid=(_GRID,),
    in_specs=[
        pl.BlockSpec((NC, _BR, F), lambda i: (0, i, 0)),
        pl.BlockSpec((NC, _BR, L), lambda i: (0, i, 0)),
        pl.BlockSpec((_BR, F), lambda i: (i, 0)),
    ],
    out_specs=pl.BlockSpec((_BR, F), lambda i: (i, 0)),
    out_shape=jax.ShapeDtypeStruct((NP, F), jnp.float32),
)


def _head_body(p_ref, w1_ref, b1_ref, w2_ref, b2_ref, d1_ref, db1_ref,
               d2_ref, db2_ref, o_ref):
    w1 = w1_ref[...]
    b1 = b1_ref[...]
    a = [jnp.maximum(jnp.dot(p_ref[:, t, :], w1,
                             preferred_element_type=jnp.float32) + b1, 0.0)
         for t in range(K)]
    m = [jnp.maximum(a[2 * u], a[2 * u + 1]) for u in range(K // 2)]
    b2 = b2_ref[...]
    c2 = []
    for t in range(11):
        acc = b2
        for tau in range(5):
            acc = acc + jnp.dot(m[t + tau], w2_ref[tau],
                                preferred_element_type=jnp.float32)
        c2.append(jnp.maximum(acc, 0.0))
    acc = db1_ref[...]
    for t in range(11):
        acc = acc + jnp.dot(c2[t], d1_ref[t],
                            preferred_element_type=jnp.float32)
    hdd = jnp.maximum(acc, 0.0)
    o_ref[...] = jnp.dot(hdd, d2_ref[...],
                         preferred_element_type=jnp.float32) + db2_ref[...]


_head = pl.pallas_call(
    _head_body,
    out_shape=jax.ShapeDtypeStruct((B, 10), jnp.float32),
)


def kernel(x, edge_index, batch, W0, b0, W1, b1, W2, b2, W3, b3, cw1, cb1,
           cw2, cb2, dw1, db1, dw2, db2):
    f32 = jnp.float32
    src = edge_index[0]
    dst = edge_index[1]
    pad_idx = N + (jnp.arange(EP - E, dtype=jnp.int32) % 16)
    srcp = jnp.concatenate([src, pad_idx])
    dstp = jnp.concatenate([dst, pad_idx])
    xp = jnp.pad(x, ((0, NP - N), (0, 0)))
    z32 = jnp.zeros((NP, F), f32)
    z16 = jnp.zeros((NP, L), f32)

    xl0 = _k0(xp, W0.T, b0.reshape(1, F))
    p1, pdeg = _agg_deg(xl0, srcp, dstp, z32, z16)
    h1, xl1 = _klayer(p1, pdeg, xl0, W1.T, b1.reshape(1, F))
    p2 = _agg(xl1, srcp, dstp, z32)
    h2, xl2 = _klayer(p2, pdeg, xl1, W2.T, b2.reshape(1, F))
    p3 = _agg(xl2, srcp, dstp, z32)
    w3pt = jnp.pad(W3.T, ((0, 0), (0, F - 1)))
    b3p = jnp.pad(b3, (0, F - 1)).reshape(1, F)
    h3, xl3p = _klayer(p3, pdeg, xl2, w3pt, b3p)
    p4 = _agg(xl3p, srcp, dstp, z32)
    h4f = _k4(p4, pdeg, xl3p)

    xcat = jnp.concatenate([h1, h2, h3, h4f], axis=1)
    keys2 = h4f[:, 0].reshape(KROWS, L)
    batch2 = jnp.pad(batch, (0, NP - N),
                     constant_values=B).reshape(KROWS, L)
    pooled = _pool(keys2, batch2, xcat)

    w1r = jnp.pad(cw1[:, 0, :].T, ((0, D - 97), (0, 0)))
    w2r = jnp.transpose(cw2, (2, 1, 0))
    d1r = jnp.transpose(dw1.reshape(128, 32, 11), (2, 1, 0))
    return _head(pooled, w1r, cb1.reshape(1, 16), w2r, cb2.reshape(1, 32),
                 d1r, db1.reshape(1, 128), dw2.T, db2.reshape(1, 10))


# speedup vs baseline: 28.9456x; 1.9636x over previous
import functools

import jax
import jax.numpy as jnp
from jax import lax
from jax.experimental import pallas as pl
from jax.experimental.pallas import tpu as pltpu
from jax.experimental.pallas import tpu_sc as plsc

N = 10000
NP = 10240
E = 320000
F = 32
D = 128
NC, NS, L = 2, 16, 16
NW = NC * NS
CH = 128
CPT = 79
EP = CPT * NW * CH
RPS = NP // NS
KROWS = NP // L
B = 64
K = 30

_mesh = plsc.VectorSubcoreMesh(core_axis_name="c", subcore_axis_name="s",
                               num_cores=NC, num_subcores=NS)
_sc_params = pltpu.CompilerParams(use_tc_tiling_on_sc=False,
                                  needs_layout_passes=False)


def _iota16():
    return lax.broadcasted_iota(jnp.int32, (L,), 0)


def _agg_body(with_deg):
    def body(xl_hbm, src2_hbm, dst2_hbm, z32_hbm, *rest):
        if with_deg:
            (z16_hbm, p_hbm, pd_hbm, acc, sbuf, dbuf, gb0, gb1, gs0, gs1,
             ss0, ss1, accd, ones, ds0, ds1) = rest
            dsem = (ds0, ds1)
        else:
            p_hbm, acc, sbuf, dbuf, gb0, gb1, gs0, gs1, ss0, ss1 = rest
        gbuf = (gb0, gb1)
        gsem = (gs0, gs1)
        ssem = (ss0, ss1)
        c = lax.axis_index("c")
        s = lax.axis_index("s")
        wid = c * NS + s
        r0 = s * RPS
        pltpu.sync_copy(z32_hbm.at[pl.ds(r0, RPS)], acc.at[pl.ds(r0, RPS)])
        if with_deg:
            pltpu.sync_copy(z16_hbm.at[pl.ds(r0, RPS)], accd.at[pl.ds(r0, RPS)])

            def oinit(r, _):
                ones[r, :] = jnp.ones((L,), jnp.float32)
                return 0

            lax.fori_loop(0, CH, oinit, 0)
        row0 = wid * CPT
        pltpu.sync_copy(src2_hbm.at[pl.ds(row0, CPT)], sbuf)
        pltpu.sync_copy(dst2_hbm.at[pl.ds(row0, CPT)], dbuf)
        plsc.subcore_barrier()

        hg = [None] * CPT
        hs = [None] * CPT
        hd = [None] * CPT
        for k in range(CPT):
            p = k & 1
            if k >= 2:
                hs[k - 2].wait()
                if with_deg:
                    hd[k - 2].wait()
            hg[k] = pltpu.async_copy(xl_hbm.at[sbuf.at[k]], gbuf[p], gsem[p])
            if k >= 1:
                q = (k - 1) & 1
                hg[k - 1].wait()
                hs[k - 1] = pltpu.async_copy(gbuf[q], acc.at[dbuf.at[k - 1]],
                                             ssem[q], add=True)
                if with_deg:
                    hd[k - 1] = pltpu.async_copy(ones, accd.at[sbuf.at[k - 1]],
                                                 dsem[q], add=True)
        k = CPT - 1
        p = k & 1
        hg[k].wait()
        hs[k] = pltpu.async_copy(gbuf[p], acc.at[dbuf.at[k]], ssem[p],
                                 add=True)
        hs[k - 1].wait()
        hs[k].wait()
        if with_deg:
            hd[k] = pltpu.async_copy(ones, accd.at[sbuf.at[k]], dsem[p],
                                     add=True)
            hd[k - 1].wait()
            hd[k].wait()
        plsc.subcore_barrier()
        pltpu.sync_copy(acc.at[pl.ds(r0, RPS)], p_hbm.at[c, pl.ds(r0, RPS)])
        if with_deg:
            pltpu.sync_copy(accd.at[pl.ds(r0, RPS)],
                            pd_hbm.at[c, pl.ds(r0, RPS)])

    return body


_agg_scratch = [
    pltpu.VMEM_SHARED((NP, F), jnp.float32),
    pltpu.VMEM((CPT, CH), jnp.int32),
    pltpu.VMEM((CPT, CH), jnp.int32),
    pltpu.VMEM((CH, F), jnp.float32),
    pltpu.VMEM((CH, F), jnp.float32),
    pltpu.SemaphoreType.DMA,
    pltpu.SemaphoreType.DMA,
    pltpu.SemaphoreType.DMA,
    pltpu.SemaphoreType.DMA,
]

_agg_deg = pl.kernel(
    _agg_body(True),
    out_type=(jax.ShapeDtypeStruct((NC, NP, F), jnp.float32),
              jax.ShapeDtypeStruct((NC, NP, L), jnp.float32)),
    mesh=_mesh,
    compiler_params=_sc_params,
    scratch_types=_agg_scratch + [pltpu.VMEM_SHARED((NP, L), jnp.float32),
                                  pltpu.VMEM((CH, L), jnp.float32),
                                  pltpu.SemaphoreType.DMA,
                                  pltpu.SemaphoreType.DMA],
)

_agg = pl.kernel(
    _agg_body(False),
    out_type=jax.ShapeDtypeStruct((NC, NP, F), jnp.float32),
    mesh=_mesh,
    compiler_params=_sc_params,
    scratch_types=_agg_scratch,
)


def _pool_body(keys_hbm, batch_hbm, xcat_hbm, out_hbm, kbuf, bbuf, idxbuf,
               rows, sem):
    c = lax.axis_index("c")
    s = lax.axis_index("s")
    wid = c * NS + s
    pltpu.sync_copy(keys_hbm, kbuf)
    pltpu.sync_copy(batch_hbm, bbuf)
    it = _iota16()
    lane0 = it == 0
    neginf = jnp.full((L,), -jnp.inf, jnp.float32)
    for gi in range(2):
        g = 2 * wid + gi

        def cstep(r, carry):
            bc, bs = carry
            bvec = bbuf[r]
            bc = bc + jnp.where(bvec == g, 1, 0).astype(jnp.int32)
            bs = bs + jnp.where(bvec < g, 1, 0).astype(jnp.int32)
            return bc, bs

        zeros16i = jnp.zeros((L,), jnp.int32)
        bc, bs = lax.fori_loop(0, KROWS, cstep, (zeros16i, zeros16i))
        cnt = jnp.sum(bc)
        start = jnp.sum(bs)
        end = start + cnt
        rlo = start // L
        rhi = (end + L - 1) // L
        dum = N + ((it + 2 * wid) & 15)
        idxbuf[pl.ds(0, L)] = dum
        idxbuf[pl.ds(L, L)] = dum

        def tstep(t, _):
            def rstep(r, vc):
                vb, ib = vc
                v = kbuf[r]
                e = r * L + it
                vm = jnp.where((e >= start) & (e < end), v, -jnp.inf)
                upd = vm > vb
                return jnp.where(upd, vm, vb), jnp.where(upd, e, ib)

            vb, ib = lax.fori_loop(rlo, rhi, rstep,
                                   (neginf, jnp.full((L,), 2**30, jnp.int32)))
            gmax = jnp.max(vb)
            sel = jnp.min(jnp.where(vb == gmax, ib, 2**30))
            valid = t < cnt
            node = jnp.where(valid, sel, N + ((2 * wid + t) & 15))
            plsc.store_scatter(idxbuf, [jnp.broadcast_to(t, (L,))],
                               jnp.broadcast_to(node, (L,)), mask=lane0)
            plsc.store_scatter(
                kbuf,
                [jnp.broadcast_to(sel // L, (L,)),
                 jnp.broadcast_to(sel % L, (L,))],
                neginf, mask=lane0 & jnp.broadcast_to(valid, (L,)))
            return 0

        lax.fori_loop(0, K, tstep, 0)
        pltpu.async_copy(xcat_hbm.at[idxbuf], rows, sem).wait()

        def zstep(t, _):
            for cc in range(8):
                rows[t, pl.ds(cc * L, L)] = jnp.zeros((L,), jnp.float32)
            return 0

        lax.fori_loop(jnp.minimum(cnt, K), K, zstep, 0)
        pltpu.sync_copy(rows.at[pl.ds(0, K)], out_hbm.at[g])


_pool = pl.kernel(
    _pool_body,
    out_type=jax.ShapeDtypeStruct((B, K, D), jnp.float32),
    mesh=_mesh,
    compiler_params=_sc_params,
    scratch_types=[
        pltpu.VMEM((KROWS, L), jnp.float32),
        pltpu.VMEM((KROWS, L), jnp.int32),
        pltpu.VMEM((2 * L,), jnp.int32),
        pltpu.VMEM((2 * L, D), jnp.float32),
        pltpu.SemaphoreType.DMA,
    ],
)


_GRID = 8
_BR = NP // _GRID


def _k0_body(x_ref, w_ref, b_ref, o_ref):
    o_ref[...] = jnp.dot(x_ref[...], w_ref[...],
                         preferred_element_type=jnp.float32) + b_ref[...]


_k0 = pl.pallas_call(
    _k0_body,
    grid=(_GRID,),
    in_specs=[
        pl.BlockSpec((_BR, D), lambda i: (i, 0)),
        pl.BlockSpec((D, F), lambda i: (0, 0)),
        pl.BlockSpec((1, F), lambda i: (0, 0)),
    ],
    out_specs=pl.BlockSpec((_BR, F), lambda i: (i, 0)),
    out_shape=jax.ShapeDtypeStruct((NP, F), jnp.float32),
)


def _klayer_body(p_ref, pd_ref, xl_ref, w_ref, b_ref, h_ref, o_ref):
    pd = pd_ref[0] + pd_ref[1]
    dinv = 1.0 / (pd[:, 0:1] + 1.0)
    h = jnp.tanh((p_ref[0] + p_ref[1] + xl_ref[...]) * dinv)
    h_ref[...] = h
    o_ref[...] = jnp.dot(h, w_ref[...],
                         preferred_element_type=jnp.float32) + b_ref[...]


_klayer = pl.pallas_call(
    _klayer_body,
    grid=(_GRID,),
    in_specs=[
        pl.BlockSpec((NC, _BR, F), lambda i: (0, i, 0)),
        pl.BlockSpec((NC, _BR, L), lambda i: (0, i, 0)),
        pl.BlockSpec((_BR, F), lambda i: (i, 0)),
        pl.BlockSpec((F, F), lambda i: (0, 0)),
        pl.BlockSpec((1, F), lambda i: (0, 0)),
    ],
    out_specs=[
        pl.BlockSpec((_BR, F), lambda i: (i, 0)),
        pl.BlockSpec((_BR, F), lambda i: (i, 0)),
    ],
    out_shape=[jax.ShapeDtypeStruct((NP, F), jnp.float32),
               jax.ShapeDtypeStruct((NP, F), jnp.float32)],
)


def _k4_body(p_ref, pd_ref, xl_ref, h_ref):
    pd = pd_ref[0] + pd_ref[1]
    dinv = 1.0 / (pd[:, 0:1] + 1.0)
    h_ref[...] = jnp.tanh((p_ref[0] + p_ref[1] + xl_ref[...]) * dinv)


_k4 = pl.pallas_call(
    _k4_body,
    grid=(_GRID,),
    in_specs=[
        pl.BlockSpec((NC, _BR, F), lambda i: (0, i, 0)),
        pl.BlockSpec((NC, _BR, L), lambda i: (0, i, 0)),
        pl.BlockSpec((_BR, F), lambda i: (i, 0)),
    ],
    out_specs=pl.BlockSpec((_BR, F), lambda i: (i, 0)),
    out_shape=jax.ShapeDtypeStruct((NP, F), jnp.float32),
)


def _head_body(p_ref, w1_ref, b1_ref, w2_ref, b2_ref, d1_ref, db1_ref,
               d2_ref, db2_ref, o_ref):
    w1 = w1_ref[...]
    b1 = b1_ref[...]
    a = [jnp.maximum(jnp.dot(p_ref[:, t, :], w1,
                             preferred_element_type=jnp.float32) + b1, 0.0)
         for t in range(K)]
    m = [jnp.maximum(a[2 * u], a[2 * u + 1]) for u in range(K // 2)]
    b2 = b2_ref[...]
    c2 = []
    for t in range(11):
        acc = b2
        for tau in range(5):
            acc = acc + jnp.dot(m[t + tau], w2_ref[tau],
                                preferred_element_type=jnp.float32)
        c2.append(jnp.maximum(acc, 0.0))
    acc = db1_ref[...]
    for t in range(11):
        acc = acc + jnp.dot(c2[t], d1_ref[t],
                            preferred_element_type=jnp.float32)
    hdd = jnp.maximum(acc, 0.0)
    o_ref[...] = jnp.dot(hdd, d2_ref[...],
                         preferred_element_type=jnp.float32) + db2_ref[...]


_head = pl.pallas_call(
    _head_body,
    out_shape=jax.ShapeDtypeStruct((B, 10), jnp.float32),
)


def kernel(x, edge_index, batch, W0, b0, W1, b1, W2, b2, W3, b3, cw1, cb1,
           cw2, cb2, dw1, db1, dw2, db2):
    f32 = jnp.float32
    src = edge_index[0]
    dst = edge_index[1]
    pad_idx = N + (jnp.arange(EP - E, dtype=jnp.int32) % 16)
    srcp = jnp.concatenate([src, pad_idx]).reshape(NW * CPT, CH)
    dstp = jnp.concatenate([dst, pad_idx]).reshape(NW * CPT, CH)
    xp = jnp.pad(x, ((0, NP - N), (0, 0)))
    z32 = jnp.zeros((NP, F), f32)
    z16 = jnp.zeros((NP, L), f32)

    xl0 = _k0(xp, W0.T, b0.reshape(1, F))
    p1, pdeg = _agg_deg(xl0, srcp, dstp, z32, z16)
    h1, xl1 = _klayer(p1, pdeg, xl0, W1.T, b1.reshape(1, F))
    p2 = _agg(xl1, srcp, dstp, z32)
    h2, xl2 = _klayer(p2, pdeg, xl1, W2.T, b2.reshape(1, F))
    p3 = _agg(xl2, srcp, dstp, z32)
    w3pt = jnp.pad(W3.T, ((0, 0), (0, F - 1)))
    b3p = jnp.pad(b3, (0, F - 1)).reshape(1, F)
    h3, xl3p = _klayer(p3, pdeg, xl2, w3pt, b3p)
    p4 = _agg(xl3p, srcp, dstp, z32)
    h4f = _k4(p4, pdeg, xl3p)

    xcat = jnp.concatenate([h1, h2, h3, h4f], axis=1)
    keys2 = h4f[:, 0].reshape(KROWS, L)
    batch2 = jnp.pad(batch, (0, NP - N),
                     constant_values=B).reshape(KROWS, L)
    pooled = _pool(keys2, batch2, xcat)

    w1r = jnp.pad(cw1[:, 0, :].T, ((0, D - 97), (0, 0)))
    w2r = jnp.transpose(cw2, (2, 1, 0))
    d1r = jnp.transpose(dw1.reshape(128, 32, 11), (2, 1, 0))
    return _head(pooled, w1r, cb1.reshape(1, 16), w2r, cb2.reshape(1, 32),
                 d1r, db1.reshape(1, 128), dw2.T, db2.reshape(1, 10))
